# baseline (speedup 1.0000x reference)
import jax
import jax.numpy as jnp
from jax import lax
from jax.experimental import pallas as pl
from jax.experimental.pallas import tpu as pltpu
from jax.experimental.pallas import tpu_sc as plsc

DIN = 256
D = 256
HALF = 128
N = 4096
CH = 8
L_LEVELS = 8
NSUB = 16
NPS = N // NSUB
CHUNK_N = 16
PAIRS = CHUNK_N * CH
CHUNKS = NPS // CHUNK_N
TAB_ROWS = N + 512
SEG = 16
HSEG = HALF // SEG


def _wx_body(x_ref, w_ref, b_ref, wf_ref, wiuo_ref):
    r = (jnp.dot(x_ref[...], w_ref[...], preferred_element_type=jnp.float32)
         + b_ref[...])
    wf_ref[0, 0] = r[:, :HALF]
    wf_ref[0, 1] = r[:, HALF:2 * HALF]
    wiuo_ref[0] = r[:, D:]


def _wx_matmul(x2, W_kernel, W_bias):
    M = x2.shape[0]
    BM = 1024
    BPL = N // BM
    return pl.pallas_call(
        _wx_body,
        grid=(M // BM,),
        in_specs=[
            pl.BlockSpec((BM, DIN), lambda i: (i, 0)),
            pl.BlockSpec((DIN, 4 * D), lambda i: (0, 0)),
            pl.BlockSpec((1, 4 * D), lambda i: (0, 0)),
        ],
        out_specs=[
            pl.BlockSpec((1, 2, BM, HALF), lambda i: (i // BPL, 0, i % BPL, 0)),
            pl.BlockSpec((1, BM, 3 * D), lambda i: (i // BPL, i % BPL, 0)),
        ],
        out_shape=[
            jax.ShapeDtypeStruct((L_LEVELS, 2, N, HALF), jnp.float32),
            jax.ShapeDtypeStruct((L_LEVELS, N, 3 * D), jnp.float32),
        ],
    )(x2, W_kernel, W_bias.reshape(1, 4 * D))


def _level_body(wxr_ref, scfo_ref, uiuo_ref, uf_ref, h_ref, c_ref,
                h2_ref, c2_ref, hu2_ref):
    i = pl.program_id(0)
    d = D
    h_sum = jnp.concatenate([scfo_ref[0, :, :HALF], scfo_ref[1, :, :HALF]],
                            axis=1)
    fco = jnp.concatenate([scfo_ref[0, :, HALF:], scfo_ref[1, :, HALF:]],
                          axis=1)
    iuo = jnp.dot(h_sum, uiuo_ref[...], preferred_element_type=jnp.float32)
    wxr = wxr_ref[...]
    gi = jax.nn.sigmoid(iuo[:, :d] + wxr[:, :d])
    gu = jnp.tanh(iuo[:, d:2 * d] + wxr[:, d:2 * d])
    go = jax.nn.sigmoid(iuo[:, 2 * d:] + wxr[:, 2 * d:])
    new_c = gi * gu + fco
    new_h = go * jnp.tanh(new_c)
    hu = jnp.dot(new_h, uf_ref[...], preferred_element_type=jnp.float32)
    h_ref[...] = new_h
    c_ref[...] = new_c
    live = (i < 8).astype(jnp.float32)
    h2_ref[0] = new_h[:, :HALF] * live
    h2_ref[1] = new_h[:, HALF:] * live
    c2_ref[0] = new_c[:, :HALF] * live
    c2_ref[1] = new_c[:, HALF:] * live
    hu2_ref[0] = hu[:, :HALF] * live
    hu2_ref[1] = hu[:, HALF:] * live


def _tc_level(wxr_t, scfo, uiuo, uf):
    BN = 512
    tab = jax.ShapeDtypeStruct((2, TAB_ROWS, HALF), jnp.float32)
    return pl.pallas_call(
        _level_body,
        grid=(TAB_ROWS // BN,),
        in_specs=[
            pl.BlockSpec((BN, 3 * D), lambda i: (jnp.minimum(i, 7), 0)),
            pl.BlockSpec((2, BN, 2 * HALF), lambda i: (0, jnp.minimum(i, 7), 0)),
            pl.BlockSpec((D, 3 * D), lambda i: (0, 0)),
            pl.BlockSpec((D, D), lambda i: (0, 0)),
        ],
        out_specs=[
            pl.BlockSpec((BN, D), lambda i: (jnp.minimum(i, 7), 0)),
            pl.BlockSpec((BN, D), lambda i: (jnp.minimum(i, 7), 0)),
            pl.BlockSpec((2, BN, HALF), lambda i: (0, i, 0)),
            pl.BlockSpec((2, BN, HALF), lambda i: (0, i, 0)),
            pl.BlockSpec((2, BN, HALF), lambda i: (0, i, 0)),
        ],
        out_shape=[
            jax.ShapeDtypeStruct((N, D), jnp.float32),
            jax.ShapeDtypeStruct((N, D), jnp.float32),
            tab, tab, tab,
        ],
    )(wxr_t, scfo, uiuo, uf)


def _sc_body(h2_hbm, c2_hbm, hu2_hbm, safe_hbm, wf_hbm, out_hbm,
             idx_v, rows_v, wf_v, acc_v, sem_g0, sem_g1, sem_o0, sem_o1):
    cc = lax.axis_index("c")
    sid = lax.axis_index("s")
    nbase = sid * NPS
    sem_g = (sem_g0, sem_g1)
    sem_o = (sem_o0, sem_o1)
    tabs = (h2_hbm, c2_hbm, hu2_hbm)

    pltpu.sync_copy(safe_hbm.at[sid], idx_v)
    off = cc * TAB_ROWS
    for ch in range(CHUNKS):
        for j in range(PAIRS // SEG):
            sl = pl.ds(j * SEG, SEG)
            idx_v[ch, sl] = idx_v[ch, sl] + off

    def start_gather(ch, b):
        for part in range(3):
            pltpu.async_copy(
                tabs[part].at[idx_v.at[ch]],
                rows_v.at[b, pl.ds(part * PAIRS, PAIRS)], sem_g[b])
        pltpu.async_copy(
            wf_hbm.at[cc, pl.ds(nbase + ch * CHUNK_N, CHUNK_N)],
            wf_v.at[b], sem_g[b])

    start_gather(0, 0)

    def compute_chunk(ch, b):
        nxt = ch + 1

        @pl.when(nxt < CHUNKS)
        def _():
            start_gather(nxt, b ^ 1)

        for part in range(3):
            pltpu.make_async_copy(
                tabs[part].at[idx_v.at[ch]],
                rows_v.at[b, pl.ds(part * PAIRS, PAIRS)], sem_g[b]).wait()
        pltpu.make_async_copy(
            wf_hbm.at[cc, pl.ds(nbase + ch * CHUNK_N, CHUNK_N)],
            wf_v.at[b], sem_g[b]).wait()

        @pl.when(ch >= 2)
        def _():
            pltpu.make_async_copy(
                acc_v.at[b],
                out_hbm.at[cc, pl.ds(nbase + (ch - 2) * CHUNK_N, CHUNK_N)],
                sem_o[b]).wait()

        def node_body(n, carry):
            p0 = n * CH
            for jg in range(0, HSEG, 4):
                G = list(range(jg, jg + 4))
                wf4 = [wf_v[b, n, pl.ds(j * SEG, SEG)] for j in G]
                acch = [rows_v[b, p0, pl.ds(j * SEG, SEG)] for j in G]
                u4 = [rows_v[b, 2 * PAIRS + p0, pl.ds(j * SEG, SEG)]
                      for j in G]
                c4 = [rows_v[b, PAIRS + p0, pl.ds(j * SEG, SEG)] for j in G]
                e4 = [jnp.exp(wf4[i] + u4[i]) for i in range(4)]
                accf = [c4[i] / (1.0 + e4[i]) for i in range(4)]
                for k in range(1, CH):
                    p = p0 + k
                    h2 = [rows_v[b, p, pl.ds(j * SEG, SEG)] for j in G]
                    u2 = [rows_v[b, 2 * PAIRS + p, pl.ds(j * SEG, SEG)]
                          for j in G]
                    c2 = [rows_v[b, PAIRS + p, pl.ds(j * SEG, SEG)]
                          for j in G]
                    acch = [acch[i] + h2[i] for i in range(4)]
                    e2 = [jnp.exp(wf4[i] + u2[i]) for i in range(4)]
                    s2 = [c2[i] / (1.0 + e2[i]) for i in range(4)]
                    accf = [accf[i] + s2[i] for i in range(4)]
                for i, j in enumerate(G):
                    acc_v[b, n, pl.ds(j * SEG, SEG)] = acch[i]
                    acc_v[b, n, pl.ds(HALF + j * SEG, SEG)] = accf[i]
            return carry

        lax.fori_loop(0, CHUNK_N, node_body, 0)
        pltpu.async_copy(
            acc_v.at[b],
            out_hbm.at[cc, pl.ds(nbase + ch * CHUNK_N, CHUNK_N)],
            sem_o[b])

    def pair_body(c2i, carry):
        for b in range(2):
            compute_chunk(c2i * 2 + b, b)
        return carry

    lax.fori_loop(0, CHUNKS // 2, pair_body, 0)
    for b in range(2):
        pltpu.make_async_copy(
            acc_v.at[b],
            out_hbm.at[cc, pl.ds(nbase + (CHUNKS - 2 + b) * CHUNK_N,
                                 CHUNK_N)],
            sem_o[b]).wait()


_sc_gather = pl.kernel(
    _sc_body,
    out_type=jax.ShapeDtypeStruct((2, N, 2 * HALF), jnp.float32),
    mesh=plsc.VectorSubcoreMesh(core_axis_name="c", subcore_axis_name="s"),
    scratch_types=[
        pltpu.VMEM((CHUNKS, PAIRS), jnp.int32),
        pltpu.VMEM((2, 3 * PAIRS, HALF), jnp.float32),
        pltpu.VMEM((2, CHUNK_N, HALF), jnp.float32),
        pltpu.VMEM((2, CHUNK_N, 2 * HALF), jnp.float32),
        pltpu.SemaphoreType.DMA,
        pltpu.SemaphoreType.DMA,
        pltpu.SemaphoreType.DMA,
        pltpu.SemaphoreType.DMA,
    ],
)


def kernel(tensor, indices, W_kernel, W_bias, Uf_kernel, Uiuo_kernel):
    L = tensor.shape[0]
    d = D
    W_kernel = jnp.concatenate([-W_kernel[:, :d], W_kernel[:, d:]], axis=1)
    W_bias = jnp.concatenate([-W_bias[:d], W_bias[d:]])
    Uf_scaled = -Uf_kernel
    wf2, wiuo = _wx_matmul(tensor.reshape(L * N, DIN), W_kernel, W_bias)
    safe = jnp.where(indices >= 1, indices - 1, N).astype(jnp.int32)
    safe = safe.reshape(L, NSUB, CHUNKS, PAIRS)

    res_h, res_c = [], []
    h2 = c2 = hu2 = None
    for t in range(L):
        if t == 0:
            scfo = jnp.zeros((2, N, 2 * HALF), jnp.float32)
        else:
            scfo = _sc_gather(h2.reshape(2 * TAB_ROWS, HALF),
                              c2.reshape(2 * TAB_ROWS, HALF),
                              hu2.reshape(2 * TAB_ROWS, HALF),
                              safe[t], wf2[t])
        h_t, c_t, h2, c2, hu2 = _tc_level(wiuo[t], scfo, Uiuo_kernel,
                                          Uf_scaled)
        res_h.append(h_t)
        res_c.append(c_t)
    return (jnp.stack(res_h), jnp.stack(res_c))
